# Initial kernel scaffold; baseline (speedup 1.0000x reference)
import functools

import jax
import jax.numpy as jnp
from jax import lax
from jax.experimental import pallas as pl
from jax.experimental.pallas import tpu as pltpu
from jax.experimental.pallas import tpu_sc as plsc

N_NODES = 10000
D = 128
EPS = 1e-5

NC = 2
NS = 16
NW = NC * NS
CHUNK = 128

NP = 10240
ROWS_PER_TILE = NP // NS
HC = 16


def _zero_rows(buf, nrows, ncols):
    def row(i, _):
        def seg(j, _):
            buf[i, pl.ds(j * 16, 16)] = jnp.zeros((16,), jnp.float32)
            return 0
        return lax.fori_loop(0, ncols // 16, seg, 0)
    lax.fori_loop(0, nrows, row, 0)


def _make_hist(e_pad):
    chunks_per_worker = e_pad // (NW * CHUNK)
    mesh = plsc.VectorSubcoreMesh(core_axis_name="c", subcore_axis_name="s")

    @functools.partial(
        pl.kernel,
        out_type=jax.ShapeDtypeStruct((NC, NP, HC), jnp.float32),
        mesh=mesh,
        scratch_types=[
            pltpu.VMEM((CHUNK,), jnp.int32),
            pltpu.VMEM((CHUNK, HC), jnp.float32),
            pltpu.VMEM((CHUNK, HC), jnp.float32),
            pltpu.VMEM_SHARED((NP, HC), jnp.float32),
        ],
    )
    def hist(dst_hbm, out_hbm, didx, ones_v, zeros_v, acc):
        c = lax.axis_index("c")
        s = lax.axis_index("s")
        wid = s * NC + c

        def fill(i, _):
            ones_v[i, :] = jnp.ones((HC,), jnp.float32)
            zeros_v[i, :] = jnp.zeros((HC,), jnp.float32)
            return 0
        lax.fori_loop(0, CHUNK, fill, 0)
        for k in range(ROWS_PER_TILE // CHUNK):
            pltpu.sync_copy(
                zeros_v, acc.at[pl.ds(s * ROWS_PER_TILE + k * CHUNK, CHUNK)]
            )
        plsc.subcore_barrier()

        base_w = wid * chunks_per_worker * CHUNK

        def body(i, _):
            base = pl.multiple_of(base_w + i * CHUNK, CHUNK)
            pltpu.sync_copy(dst_hbm.at[pl.ds(base, CHUNK)], didx)
            pltpu.sync_copy(ones_v, acc.at[didx], add=True)
            return 0
        lax.fori_loop(0, chunks_per_worker, body, 0)
        plsc.subcore_barrier()
        pltpu.sync_copy(
            acc.at[pl.ds(s * ROWS_PER_TILE, ROWS_PER_TILE)],
            out_hbm.at[c, pl.ds(s * ROWS_PER_TILE, ROWS_PER_TILE)],
        )

    return hist


def _make_edge_agg(e_pad):
    chunks_per_worker = e_pad // (NW * CHUNK)
    mesh = plsc.VectorSubcoreMesh(core_axis_name="c", subcore_axis_name="s")

    @functools.partial(
        pl.kernel,
        out_type=jax.ShapeDtypeStruct((NC, NP, D), jnp.float32),
        mesh=mesh,
        scratch_types=[
            pltpu.VMEM((CHUNK,), jnp.int32),
            pltpu.VMEM((CHUNK,), jnp.int32),
            pltpu.VMEM((CHUNK, D), jnp.float32),
            pltpu.VMEM_SHARED((NP, D), jnp.float32),
            pltpu.SemaphoreType.DMA,
        ],
    )
    def agg(y_hbm, src_hbm, dst_hbm, out_hbm, sidx, didx, rows, acc, sem):
        c = lax.axis_index("c")
        s = lax.axis_index("s")
        wid = s * NC + c

        _zero_rows(rows, CHUNK, D)
        for k in range(ROWS_PER_TILE // CHUNK):
            pltpu.sync_copy(
                rows, acc.at[pl.ds(s * ROWS_PER_TILE + k * CHUNK, CHUNK)]
            )
        plsc.subcore_barrier()

        base_w = wid * chunks_per_worker * CHUNK

        def body(i, _):
            base = pl.multiple_of(base_w + i * CHUNK, CHUNK)
            pltpu.sync_copy(src_hbm.at[pl.ds(base, CHUNK)], sidx)
            pltpu.sync_copy(dst_hbm.at[pl.ds(base, CHUNK)], didx)
            pltpu.async_copy(y_hbm.at[sidx], rows, sem).wait()
            pltpu.sync_copy(rows, acc.at[didx], add=True)
            return 0
        lax.fori_loop(0, chunks_per_worker, body, 0)
        plsc.subcore_barrier()
        pltpu.sync_copy(
            acc.at[pl.ds(s * ROWS_PER_TILE, ROWS_PER_TILE)],
            out_hbm.at[c, pl.ds(s * ROWS_PER_TILE, ROWS_PER_TILE)],
        )

    return agg


def _dis_from_degpart(degpart_ref):
    deg = degpart_ref[0][:, 0:1] + degpart_ref[1][:, 0:1] + 1.0
    return lax.rsqrt(deg)


def _prep_body(degpart_ref, x_ref, w_ref, xw_ref, y_ref):
    dis = _dis_from_degpart(degpart_ref)
    xw = jnp.dot(x_ref[...], w_ref[...], preferred_element_type=jnp.float32)
    xw_ref[...] = xw
    y_ref[...] = xw * dis


def _layer_body(acc_ref, xw_ref, degpart_ref, b_ref, g_ref, bt_ref, w_ref,
                xw2_ref, y2_ref):
    dis = _dis_from_degpart(degpart_ref)
    xw = xw_ref[...]
    h = dis * (acc_ref[0] + acc_ref[1]) + (dis * dis) * xw + b_ref[...]
    hr = h[:N_NODES]
    mean = jnp.mean(hr, axis=0, keepdims=True)
    var = jnp.mean((hr - mean) ** 2, axis=0, keepdims=True)
    hbn = g_ref[...] * (h - mean) * lax.rsqrt(var + EPS) + bt_ref[...]
    xw2 = jnp.dot(hbn, w_ref[...], preferred_element_type=jnp.float32)
    xw2_ref[...] = xw2
    y2_ref[...] = xw2 * dis


def _final_body(acc_ref, xw_ref, degpart_ref, b_ref, g_ref, bt_ref, w_ref,
                bfc_ref, out_ref):
    dis = _dis_from_degpart(degpart_ref)
    xw = xw_ref[...]
    h = dis * (acc_ref[0] + acc_ref[1]) + (dis * dis) * xw + b_ref[...]
    hr = h[:N_NODES]
    mean = jnp.mean(hr, axis=0, keepdims=True)
    var = jnp.mean((hr - mean) ** 2, axis=0, keepdims=True)
    hbn = g_ref[...] * (h - mean) * lax.rsqrt(var + EPS) + bt_ref[...]
    out_ref[...] = (
        jnp.dot(hbn, w_ref[...], preferred_element_type=jnp.float32)
        + bfc_ref[...]
    )


_f32 = jnp.float32


def _prep(degpart, x_p, W1):
    return pl.pallas_call(
        _prep_body,
        out_shape=(
            jax.ShapeDtypeStruct((NP, D), _f32),
            jax.ShapeDtypeStruct((NP, D), _f32),
        ),
    )(degpart, x_p, W1)


def _layer(acc, xw, degpart, b, g, bt, W2):
    return pl.pallas_call(
        _layer_body,
        out_shape=(
            jax.ShapeDtypeStruct((NP, D), _f32),
            jax.ShapeDtypeStruct((NP, D), _f32),
        ),
    )(acc, xw, degpart, b, g, bt, W2)


def _final(acc, xw, degpart, b, g, bt, Wfc, bfc):
    return pl.pallas_call(
        _final_body,
        out_shape=jax.ShapeDtypeStruct((NP, D), _f32),
    )(acc, xw, degpart, b, g, bt, Wfc, bfc)


def kernel(x, edge_index, W1, b1, g1, bt1, W2, b2, g2, bt2, Wfc, bfc):
    src = edge_index[0].astype(jnp.int32)
    dst = edge_index[1].astype(jnp.int32)
    n_edges = src.shape[0]
    cpw = -(-n_edges // (NW * CHUNK))
    e_pad = cpw * NW * CHUNK
    pad = e_pad - n_edges
    src_p = jnp.concatenate([src, jnp.full((pad,), N_NODES, jnp.int32)])
    dst_p = jnp.concatenate([dst, jnp.full((pad,), N_NODES, jnp.int32)])
    x_p = jnp.concatenate([x, jnp.zeros((NP - N_NODES, D), x.dtype)])

    b1r, g1r, bt1r = (v.reshape(1, D) for v in (b1, g1, bt1))
    b2r, g2r, bt2r = (v.reshape(1, D) for v in (b2, g2, bt2))
    bfcr = bfc.reshape(1, D)

    hist = _make_hist(e_pad)
    agg = _make_edge_agg(e_pad)

    degpart = hist(dst_p)
    xw1, y1 = _prep(degpart, x_p, W1)
    acc1 = agg(y1, src_p, dst_p)
    xw2, y2 = _layer(acc1, xw1, degpart, b1r, g1r, bt1r, W2)
    acc2 = agg(y2, src_p, dst_p)
    out_full = _final(acc2, xw2, degpart, b2r, g2r, bt2r, Wfc, bfcr)
    return out_full[:N_NODES]


# speedup vs baseline: 11.5571x; 11.5571x over previous
import functools

import jax
import jax.numpy as jnp
from jax import lax
from jax.experimental import pallas as pl
from jax.experimental.pallas import tpu as pltpu
from jax.experimental.pallas import tpu_sc as plsc

N_NODES = 10000
D = 128
EPS = 1e-5

NC = 2
NS = 16
NW = NC * NS
CHUNK = 128

NP = 10240
ROWS_PER_TILE = NP // NS


def _zero_rows(buf, nrows, ncols):
    def row(i, _):
        def seg(j, _):
            buf[i, pl.ds(j * 16, 16)] = jnp.zeros((16,), jnp.float32)
            return 0
        return lax.fori_loop(0, ncols // 16, seg, 0)
    lax.fori_loop(0, nrows, row, 0)


def _make_hist(e_pad):
    chunks_per_worker = e_pad // (NW * CHUNK)
    mesh = plsc.VectorSubcoreMesh(core_axis_name="c", subcore_axis_name="s")

    @functools.partial(
        pl.kernel,
        out_type=jax.ShapeDtypeStruct((NC * NP,), jnp.float32),
        mesh=mesh,
        scratch_types=[
            pltpu.VMEM((CHUNK,), jnp.int32),
            pltpu.VMEM((CHUNK,), jnp.float32),
            pltpu.VMEM((ROWS_PER_TILE,), jnp.float32),
            pltpu.VMEM_SHARED((NP,), jnp.float32),
        ],
    )
    def hist(dst_hbm, out_hbm, didx, ones_v, col_v, acc):
        c = lax.axis_index("c")
        s = lax.axis_index("s")
        wid = s * NC + c

        def fill(i, _):
            ones_v[pl.ds(i * 16, 16)] = jnp.ones((16,), jnp.float32)
            return 0
        lax.fori_loop(0, CHUNK // 16, fill, 0)

        def zero(i, _):
            col_v[pl.ds(i * 16, 16)] = jnp.zeros((16,), jnp.float32)
            return 0
        lax.fori_loop(0, ROWS_PER_TILE // 16, zero, 0)
        pltpu.sync_copy(col_v, acc.at[pl.ds(s * ROWS_PER_TILE, ROWS_PER_TILE)])
        plsc.subcore_barrier()

        base_w = wid * chunks_per_worker * CHUNK

        def body(i, _):
            base = pl.multiple_of(base_w + i * CHUNK, CHUNK)
            pltpu.sync_copy(dst_hbm.at[pl.ds(base, CHUNK)], didx)
            pltpu.sync_copy(ones_v, acc.at[didx], add=True)
            return 0
        lax.fori_loop(0, chunks_per_worker, body, 0)
        plsc.subcore_barrier()
        pltpu.sync_copy(acc.at[pl.ds(s * ROWS_PER_TILE, ROWS_PER_TILE)], col_v)
        pltpu.sync_copy(
            col_v, out_hbm.at[pl.ds(c * NP + s * ROWS_PER_TILE, ROWS_PER_TILE)]
        )

    return hist


def _make_edge_agg(e_pad):
    chunks_per_worker = e_pad // (NW * CHUNK)
    mesh = plsc.VectorSubcoreMesh(core_axis_name="c", subcore_axis_name="s")

    @functools.partial(
        pl.kernel,
        out_type=jax.ShapeDtypeStruct((NC, NP, D), jnp.float32),
        mesh=mesh,
        scratch_types=[
            pltpu.VMEM((CHUNK,), jnp.int32),
            pltpu.VMEM((CHUNK,), jnp.int32),
            pltpu.VMEM((CHUNK, D), jnp.float32),
            pltpu.VMEM_SHARED((NP, D), jnp.float32),
            pltpu.SemaphoreType.DMA,
        ],
    )
    def agg(y_hbm, src_hbm, dst_hbm, out_hbm, sidx, didx, rows, acc, sem):
        c = lax.axis_index("c")
        s = lax.axis_index("s")
        wid = s * NC + c

        _zero_rows(rows, CHUNK, D)
        for k in range(ROWS_PER_TILE // CHUNK):
            pltpu.sync_copy(
                rows, acc.at[pl.ds(s * ROWS_PER_TILE + k * CHUNK, CHUNK)]
            )
        plsc.subcore_barrier()

        base_w = wid * chunks_per_worker * CHUNK

        def body(i, _):
            base = pl.multiple_of(base_w + i * CHUNK, CHUNK)
            pltpu.sync_copy(src_hbm.at[pl.ds(base, CHUNK)], sidx)
            pltpu.sync_copy(dst_hbm.at[pl.ds(base, CHUNK)], didx)
            pltpu.async_copy(y_hbm.at[sidx], rows, sem).wait()
            pltpu.sync_copy(rows, acc.at[didx], add=True)
            return 0
        lax.fori_loop(0, chunks_per_worker, body, 0)
        plsc.subcore_barrier()
        pltpu.sync_copy(
            acc.at[pl.ds(s * ROWS_PER_TILE, ROWS_PER_TILE)],
            out_hbm.at[c, pl.ds(s * ROWS_PER_TILE, ROWS_PER_TILE)],
        )

    return agg


def _prep_body(deg_ref, x_ref, w_ref, xw_ref, y_ref):
    dis = lax.rsqrt(deg_ref[...] + 1.0)
    xw = jnp.dot(x_ref[...], w_ref[...], preferred_element_type=jnp.float32)
    xw_ref[...] = xw
    y_ref[...] = xw * dis


def _layer_body(acc_ref, xw_ref, deg_ref, b_ref, g_ref, bt_ref, w_ref,
                xw2_ref, y2_ref):
    dis = lax.rsqrt(deg_ref[...] + 1.0)
    xw = xw_ref[...]
    h = dis * (acc_ref[0] + acc_ref[1]) + (dis * dis) * xw + b_ref[...]
    hr = h[:N_NODES]
    mean = jnp.mean(hr, axis=0, keepdims=True)
    var = jnp.mean((hr - mean) ** 2, axis=0, keepdims=True)
    hbn = g_ref[...] * (h - mean) * lax.rsqrt(var + EPS) + bt_ref[...]
    xw2 = jnp.dot(hbn, w_ref[...], preferred_element_type=jnp.float32)
    xw2_ref[...] = xw2
    y2_ref[...] = xw2 * dis


def _final_body(acc_ref, xw_ref, deg_ref, b_ref, g_ref, bt_ref, w_ref,
                bfc_ref, out_ref):
    dis = lax.rsqrt(deg_ref[...] + 1.0)
    xw = xw_ref[...]
    h = dis * (acc_ref[0] + acc_ref[1]) + (dis * dis) * xw + b_ref[...]
    hr = h[:N_NODES]
    mean = jnp.mean(hr, axis=0, keepdims=True)
    var = jnp.mean((hr - mean) ** 2, axis=0, keepdims=True)
    hbn = g_ref[...] * (h - mean) * lax.rsqrt(var + EPS) + bt_ref[...]
    out_ref[...] = (
        jnp.dot(hbn, w_ref[...], preferred_element_type=jnp.float32)
        + bfc_ref[...]
    )


_f32 = jnp.float32


def _prep(deg_col, x_p, W1):
    return pl.pallas_call(
        _prep_body,
        out_shape=(
            jax.ShapeDtypeStruct((NP, D), _f32),
            jax.ShapeDtypeStruct((NP, D), _f32),
        ),
    )(deg_col, x_p, W1)


def _layer(acc, xw, deg_col, b, g, bt, W2):
    return pl.pallas_call(
        _layer_body,
        out_shape=(
            jax.ShapeDtypeStruct((NP, D), _f32),
            jax.ShapeDtypeStruct((NP, D), _f32),
        ),
    )(acc, xw, deg_col, b, g, bt, W2)


def _final(acc, xw, deg_col, b, g, bt, Wfc, bfc):
    return pl.pallas_call(
        _final_body,
        out_shape=jax.ShapeDtypeStruct((NP, D), _f32),
    )(acc, xw, deg_col, b, g, bt, Wfc, bfc)


def kernel(x, edge_index, W1, b1, g1, bt1, W2, b2, g2, bt2, Wfc, bfc):
    src = edge_index[0].astype(jnp.int32)
    dst = edge_index[1].astype(jnp.int32)
    n_edges = src.shape[0]
    cpw = -(-n_edges // (NW * CHUNK))
    e_pad = cpw * NW * CHUNK
    pad = e_pad - n_edges
    src_p = jnp.concatenate([src, jnp.full((pad,), N_NODES, jnp.int32)])
    dst_p = jnp.concatenate([dst, jnp.full((pad,), N_NODES, jnp.int32)])
    x_p = jnp.concatenate([x, jnp.zeros((NP - N_NODES, D), x.dtype)])

    b1r, g1r, bt1r = (v.reshape(1, D) for v in (b1, g1, bt1))
    b2r, g2r, bt2r = (v.reshape(1, D) for v in (b2, g2, bt2))
    bfcr = bfc.reshape(1, D)

    hist = _make_hist(e_pad)
    agg = _make_edge_agg(e_pad)

    degflat = hist(dst_p)
    deg_col = (degflat[:NP] + degflat[NP:])[:, None]

    xw1, y1 = _prep(deg_col, x_p, W1)
    acc1 = agg(y1, src_p, dst_p)
    xw2, y2 = _layer(acc1, xw1, deg_col, b1r, g1r, bt1r, W2)
    acc2 = agg(y2, src_p, dst_p)
    out_full = _final(acc2, xw2, deg_col, b2r, g2r, bt2r, Wfc, bfcr)
    return out_full[:N_NODES]
